# baseline (speedup 1.0000x reference)
import functools

import jax
import jax.numpy as jnp
from jax import lax
from jax.experimental import pallas as pl
from jax.experimental.pallas import tpu as pltpu
from jax.experimental.pallas import tpu_sc as plsc

NC = 2
NS = 16
NW = NC * NS
L = 16
CH = 50


def _zero_rows(buf, nrows, ncols):
  z = jnp.zeros((L,), jnp.float32)

  def body(i, _):
    for jj in range(ncols // L):
      buf[i, pl.ds(jj * L, L)] = z
    return 0

  lax.fori_loop(0, nrows, body, 0)


def _fill_ones(buf, nrows, ncols):
  o = jnp.ones((L,), jnp.float32)

  def body(i, _):
    for jj in range(ncols // L):
      buf[i, pl.ds(jj * L, L)] = o
    return 0

  lax.fori_loop(0, nrows, body, 0)


def _zero_slab(zsrc, zrows, acc, row0, npt):
  off = 0
  while off < npt:
    step = min(zrows, npt - off)
    pltpu.sync_copy(zsrc.at[pl.ds(0, step)],
                    acc.at[pl.ds(row0 + off, step)])
    off += step


def _make_deg_kernel(n, e):
  ring = 8
  assert e % (CH * NW) == 0 and n % (8 * NS) == 0
  nch = e // CH
  ncw = nch // NW
  assert ncw >= ring
  npt = n // NS
  zr = min(npt, CH)
  mesh = plsc.VectorSubcoreMesh(core_axis_name="c", subcore_axis_name="s")

  @functools.partial(
      pl.kernel,
      out_type=jax.ShapeDtypeStruct((NC, n, L), jnp.float32),
      mesh=mesh,
      compiler_params=pltpu.CompilerParams(use_tc_tiling_on_sc=False),
      scratch_types=[
          pltpu.VMEM_SHARED((n, L), jnp.float32),
          pltpu.VMEM((CH, L), jnp.float32),
          pltpu.VMEM((ncw, CH), jnp.int32),
      ],
  )
  def deg_kernel(dst2_hbm, degp_hbm, acc, buf, didx_all):
    c = lax.axis_index("c")
    s = lax.axis_index("s")
    w = c * NS + s
    row0 = s * npt

    pltpu.sync_copy(dst2_hbm.at[pl.ds(w * ncw, ncw)], didx_all)

    _zero_rows(buf, zr, L)
    off = 0
    while off < npt:
      step = min(zr, npt - off)
      pltpu.sync_copy(buf.at[pl.ds(0, step)],
                      acc.at[pl.ds(row0 + off, step)])
      off += step
    plsc.subcore_barrier()

    _fill_ones(buf, CH, L)

    def body(j, _):
      pltpu.sync_copy(buf, acc.at[didx_all.at[j]], add=True)
      return 0

    lax.fori_loop(0, ncw, body, 0)
    plsc.subcore_barrier()
    pltpu.sync_copy(acc.at[pl.ds(row0, npt)],
                    degp_hbm.at[c, pl.ds(row0, npt)])

  return deg_kernel


def _make_agg_kernel(n, e, d):
  assert e % (CH * NW) == 0 and n % (8 * NS) == 0 and d % L == 0
  nch = e // CH
  ncw = nch // NW
  npt = n // NS
  zr = min(npt, CH)
  ring = 8 if d <= 64 else 4
  assert ncw >= ring
  mesh = plsc.VectorSubcoreMesh(core_axis_name="c", subcore_axis_name="s")

  @functools.partial(
      pl.kernel,
      out_type=jax.ShapeDtypeStruct((NC, n, d), jnp.float32),
      mesh=mesh,
      compiler_params=pltpu.CompilerParams(use_tc_tiling_on_sc=False),
      scratch_types=[
          pltpu.VMEM_SHARED((n, d), jnp.float32),
          pltpu.VMEM((ring, CH, d), jnp.float32),
          pltpu.VMEM((ncw, CH), jnp.int32),
          pltpu.VMEM((ncw, CH), jnp.int32),
          pltpu.SemaphoreType.DMA((ring,)),
      ],
  )
  def agg_kernel(y_hbm, src2_hbm, dst2_hbm, aggp_hbm,
                 acc, rows_v, sidx_all, didx_all, gsem):
    c = lax.axis_index("c")
    s = lax.axis_index("s")
    w = c * NS + s
    row0 = s * npt

    pltpu.sync_copy(src2_hbm.at[pl.ds(w * ncw, ncw)], sidx_all)
    pltpu.sync_copy(dst2_hbm.at[pl.ds(w * ncw, ncw)], didx_all)

    zslot = rows_v.at[0]
    _zero_rows(zslot, zr, d)
    off = 0
    while off < npt:
      step = min(zr, npt - off)
      pltpu.sync_copy(zslot.at[pl.ds(0, step)],
                      acc.at[pl.ds(row0 + off, step)])
      off += step
    plsc.subcore_barrier()

    for jj in range(ring):
      pltpu.async_copy(y_hbm.at[sidx_all.at[jj]], rows_v.at[jj],
                       gsem.at[jj])

    def body(j, _):
      rb = j % ring
      pltpu.make_async_copy(y_hbm.at[sidx_all.at[j]], rows_v.at[rb],
                            gsem.at[rb]).wait()
      pltpu.sync_copy(rows_v.at[rb], acc.at[didx_all.at[j]], add=True)
      pltpu.async_copy(y_hbm.at[sidx_all.at[j + ring]], rows_v.at[rb],
                       gsem.at[rb])
      return 0

    lax.fori_loop(0, ncw - ring, body, 0)

    def tail(j, _):
      rb = j % ring
      pltpu.make_async_copy(y_hbm.at[sidx_all.at[j]], rows_v.at[rb],
                            gsem.at[rb]).wait()
      pltpu.sync_copy(rows_v.at[rb], acc.at[didx_all.at[j]], add=True)
      return 0

    lax.fori_loop(ncw - ring, ncw, tail, 0)
    plsc.subcore_barrier()
    pltpu.sync_copy(acc.at[pl.ds(row0, npt)],
                    aggp_hbm.at[c, pl.ds(row0, npt)])

  return agg_kernel


def _make_agg2_fused_kernel(n, e, dh, n_tab):
  ring = 8
  assert e % (CH * NS) == 0 and n % (8 * NS) == 0 and dh % L == 0
  nch = e // CH
  ncw = nch // NS
  assert ncw % ring == 0 and ncw >= 2 * ring
  npt = n // NS
  zr = min(npt, CH)
  assert n_tab >= n
  mesh = plsc.VectorSubcoreMesh(core_axis_name="c", subcore_axis_name="s")

  @functools.partial(
      pl.kernel,
      out_type=jax.ShapeDtypeStruct((NC, n, dh), jnp.float32),
      mesh=mesh,
      compiler_params=pltpu.CompilerParams(use_tc_tiling_on_sc=False),
      scratch_types=[
          pltpu.VMEM_SHARED((n, dh), jnp.float32),
          pltpu.VMEM((ring, CH, dh), jnp.float32),
          pltpu.VMEM((CH, dh), jnp.float32),
          pltpu.VMEM((CH, L), jnp.float32),
          pltpu.VMEM((ncw, CH), jnp.int32),
          pltpu.VMEM((ncw, CH), jnp.int32),
          pltpu.SemaphoreType.DMA((ring,)),
      ],
  )
  def agg2_kernel(y_hbm, srcs_hbm, dst2_hbm, dinv_hbm, out_hbm,
                  acc, rows_v, ybuf, dbuf, sidx_all, didx_all, gsem):
    cc = lax.axis_index("c")
    s = lax.axis_index("s")
    row0 = s * npt

    pltpu.sync_copy(srcs_hbm.at[cc, pl.ds(s * ncw, ncw)], sidx_all)
    pltpu.sync_copy(dst2_hbm.at[pl.ds(s * ncw, ncw)], didx_all)

    zslot = rows_v.at[0]
    _zero_rows(zslot, zr, dh)
    _zero_slab(zslot, zr, acc, row0, npt)
    plsc.subcore_barrier()

    for jj in range(ring):
      pltpu.async_copy(y_hbm.at[sidx_all.at[jj]], rows_v.at[jj],
                       gsem.at[jj])

    def body(j, _):
      rb = j % ring
      pltpu.make_async_copy(y_hbm.at[sidx_all.at[j]], rows_v.at[rb],
                            gsem.at[rb]).wait()
      pltpu.sync_copy(rows_v.at[rb], acc.at[didx_all.at[j]], add=True)
      pltpu.async_copy(y_hbm.at[sidx_all.at[j + ring]], rows_v.at[rb],
                       gsem.at[rb])
      return 0

    lax.fori_loop(0, ncw - ring, body, 0)

    def tail(j, _):
      rb = j % ring
      pltpu.make_async_copy(y_hbm.at[sidx_all.at[j]], rows_v.at[rb],
                            gsem.at[rb]).wait()
      pltpu.sync_copy(rows_v.at[rb], acc.at[didx_all.at[j]], add=True)
      return 0

    lax.fori_loop(ncw - ring, ncw, tail, 0)
    plsc.subcore_barrier()

    aslot = rows_v.at[1]
    off = 0
    while off < npt:
      step = min(CH, npt - off)
      pltpu.sync_copy(acc.at[pl.ds(row0 + off, step)],
                      aslot.at[pl.ds(0, step)])
      pltpu.sync_copy(y_hbm.at[pl.ds(cc * n_tab + row0 + off, step)],
                      ybuf.at[pl.ds(0, step)])
      pltpu.sync_copy(dinv_hbm.at[pl.ds(row0 + off, step)],
                      dbuf.at[pl.ds(0, step)])

      def erow(i, _):
        dv = dbuf[i, pl.ds(0, L)]
        for kk in range(dh // L):
          sl = pl.ds(kk * L, L)
          aslot[i, sl] = (aslot[i, sl] + ybuf[i, sl]) * dv
        return 0

      lax.fori_loop(0, step, erow, 0)
      pltpu.sync_copy(aslot.at[pl.ds(0, step)],
                      out_hbm.at[cc, pl.ds(row0 + off, step)])
      off += step

  return agg2_kernel


def _tc_layer1(degp0_ref, degp1_ref, x_ref, w1_ref, dinv_ref, y1_ref):
  deg = degp0_ref[...] + degp1_ref[...] + 1.0
  dinv = lax.rsqrt(deg)
  dinv_ref[...] = dinv
  xw = jnp.dot(x_ref[...], w1_ref[...], preferred_element_type=jnp.float32)
  y1_ref[...] = xw * dinv[:, 0:1]


def _tc_layer2(aggp0_ref, aggp1_ref, y1_ref, dinv_ref, w2_ref, y2_ref):
  dv = dinv_ref[...][:, 0:1]
  h = jnp.tanh((aggp0_ref[...] + aggp1_ref[...] + y1_ref[...]) * dv)
  y2_ref[0] = jnp.dot(h, w2_ref[0],
                      preferred_element_type=jnp.float32) * dv


def kernel(x, edge_index, W1, W2):
  n, f_in = x.shape
  e = edge_index.shape[1]
  h = W1.shape[1]
  cdim = W2.shape[1]
  cpad = 128
  dh2 = cpad // 2
  assert e % CH == 0
  src2 = edge_index[0].reshape(e // CH, CH)
  dst2 = edge_index[1].reshape(e // CH, CH)
  W2p = jnp.zeros((h, cpad), jnp.float32).at[:, :cdim].set(W2)
  W2s = jnp.stack([W2p[:, :dh2], W2p[:, dh2:]])
  np_pad = -(-n // (8 * NS)) * (8 * NS)
  srcs = jnp.stack([src2, src2 + np_pad])

  blk = 2000
  assert n % blk == 0
  nb = n // blk
  row_spec = lambda width: pl.BlockSpec((blk, width), lambda i: (i, 0))
  full_spec = lambda r, ccol: pl.BlockSpec((r, ccol), lambda i: (0, 0))
  row2 = lambda width: pl.BlockSpec((blk, width), lambda c, i: (i, 0))
  plane2 = lambda width: pl.BlockSpec((1, blk, width),
                                      lambda c, i: (c, i, 0))
  wcol2 = lambda r, width: pl.BlockSpec((1, r, width),
                                        lambda c, i: (c, 0, 0))

  degp = _make_deg_kernel(np_pad, e)(dst2)

  dinv, y1 = pl.pallas_call(
      _tc_layer1,
      grid=(nb,),
      in_specs=[row_spec(L), row_spec(L), row_spec(f_in), full_spec(f_in, h)],
      out_specs=[row_spec(L), row_spec(h)],
      out_shape=[
          jax.ShapeDtypeStruct((np_pad, L), jnp.float32),
          jax.ShapeDtypeStruct((n, h), jnp.float32),
      ],
  )(degp[0], degp[1], x, W1)

  aggp1 = _make_agg_kernel(np_pad, e, h)(y1, src2, dst2)

  y2 = pl.pallas_call(
      _tc_layer2,
      grid=(2, nb),
      in_specs=[row2(h), row2(h), row2(h), row2(L), wcol2(h, dh2)],
      out_specs=plane2(dh2),
      out_shape=jax.ShapeDtypeStruct((2, np_pad, dh2), jnp.float32),
  )(aggp1[0], aggp1[1], y1, dinv, W2s)

  out2 = _make_agg2_fused_kernel(np_pad, e, dh2, np_pad)(
      y2.reshape(2 * np_pad, dh2), srcs, dst2, dinv)

  return jnp.concatenate([out2[0, :n], out2[1, :n]], axis=1)[:, :cdim]


# speedup vs baseline: 1.1448x; 1.1448x over previous
import functools

import jax
import jax.numpy as jnp
from jax import lax
from jax.experimental import pallas as pl
from jax.experimental.pallas import tpu as pltpu
from jax.experimental.pallas import tpu_sc as plsc

NC = 2
NS = 16
NW = NC * NS
L = 16
CH = 50


def _zero_rows(buf, nrows, ncols):
  z = jnp.zeros((L,), jnp.float32)

  def body(i, _):
    for jj in range(ncols // L):
      buf[i, pl.ds(jj * L, L)] = z
    return 0

  lax.fori_loop(0, nrows, body, 0)


def _fill_ones(buf, nrows, ncols):
  o = jnp.ones((L,), jnp.float32)

  def body(i, _):
    for jj in range(ncols // L):
      buf[i, pl.ds(jj * L, L)] = o
    return 0

  lax.fori_loop(0, nrows, body, 0)


def _make_deg_kernel(n, e):
  ring = 8
  assert e % (CH * NW) == 0 and n % (8 * NS) == 0
  nch = e // CH
  ncw = nch // NW
  assert ncw >= ring
  npt = n // NS
  zr = min(npt, CH)
  mesh = plsc.VectorSubcoreMesh(core_axis_name="c", subcore_axis_name="s")

  @functools.partial(
      pl.kernel,
      out_type=jax.ShapeDtypeStruct((NC, n, L), jnp.float32),
      mesh=mesh,
      compiler_params=pltpu.CompilerParams(use_tc_tiling_on_sc=False),
      scratch_types=[
          pltpu.VMEM_SHARED((n, L), jnp.float32),
          pltpu.VMEM((CH, L), jnp.float32),
          pltpu.VMEM((ncw, CH), jnp.int32),
          pltpu.SemaphoreType.DMA((ring,)),
      ],
  )
  def deg_kernel(dst2_hbm, degp_hbm, acc, buf, didx_all, ssem):
    c = lax.axis_index("c")
    s = lax.axis_index("s")
    w = c * NS + s
    row0 = s * npt

    pltpu.sync_copy(dst2_hbm.at[pl.ds(w * ncw, ncw)], didx_all)

    _zero_rows(buf, zr, L)
    off = 0
    while off < npt:
      step = min(zr, npt - off)
      pltpu.sync_copy(buf.at[pl.ds(0, step)],
                      acc.at[pl.ds(row0 + off, step)])
      off += step
    plsc.subcore_barrier()

    _fill_ones(buf, CH, L)

    def body(j, _):
      pltpu.sync_copy(buf, acc.at[didx_all.at[j]], add=True)
      return 0

    lax.fori_loop(0, ncw, body, 0)
    plsc.subcore_barrier()
    pltpu.sync_copy(acc.at[pl.ds(row0, npt)],
                    degp_hbm.at[c, pl.ds(row0, npt)])

  return deg_kernel


def _make_agg_kernel(n, e, d):
  assert e % (CH * NW) == 0 and n % (8 * NS) == 0 and d % L == 0
  nch = e // CH
  ncw = nch // NW
  npt = n // NS
  zr = min(npt, CH)
  ring = 8 if d <= 64 else 4
  assert ncw >= ring
  mesh = plsc.VectorSubcoreMesh(core_axis_name="c", subcore_axis_name="s")

  @functools.partial(
      pl.kernel,
      out_type=jax.ShapeDtypeStruct((NC, n, d), jnp.float32),
      mesh=mesh,
      compiler_params=pltpu.CompilerParams(use_tc_tiling_on_sc=False),
      scratch_types=[
          pltpu.VMEM_SHARED((n, d), jnp.float32),
          pltpu.VMEM((ring, CH, d), jnp.float32),
          pltpu.VMEM((ncw, CH), jnp.int32),
          pltpu.VMEM((ncw, CH), jnp.int32),
          pltpu.SemaphoreType.DMA((ring,)),
      ],
  )
  def agg_kernel(y_hbm, src2_hbm, dst2_hbm, aggp_hbm,
                 acc, rows_v, sidx_all, didx_all, gsem):
    c = lax.axis_index("c")
    s = lax.axis_index("s")
    w = c * NS + s
    row0 = s * npt

    pltpu.sync_copy(src2_hbm.at[pl.ds(w * ncw, ncw)], sidx_all)
    pltpu.sync_copy(dst2_hbm.at[pl.ds(w * ncw, ncw)], didx_all)

    zslot = rows_v.at[0]
    _zero_rows(zslot, zr, d)
    off = 0
    while off < npt:
      step = min(zr, npt - off)
      pltpu.sync_copy(zslot.at[pl.ds(0, step)],
                      acc.at[pl.ds(row0 + off, step)])
      off += step
    plsc.subcore_barrier()

    for jj in range(ring):
      pltpu.async_copy(y_hbm.at[sidx_all.at[jj]], rows_v.at[jj],
                       gsem.at[jj])

    def body(j, _):
      rb = j % ring
      pltpu.make_async_copy(y_hbm.at[sidx_all.at[j]], rows_v.at[rb],
                            gsem.at[rb]).wait()
      pltpu.sync_copy(rows_v.at[rb], acc.at[didx_all.at[j]], add=True)
      pltpu.async_copy(y_hbm.at[sidx_all.at[j + ring]], rows_v.at[rb],
                       gsem.at[rb])
      return 0

    lax.fori_loop(0, ncw - ring, body, 0)

    def tail(j, _):
      rb = j % ring
      pltpu.make_async_copy(y_hbm.at[sidx_all.at[j]], rows_v.at[rb],
                            gsem.at[rb]).wait()
      pltpu.sync_copy(rows_v.at[rb], acc.at[didx_all.at[j]], add=True)
      return 0

    lax.fori_loop(ncw - ring, ncw, tail, 0)
    plsc.subcore_barrier()
    pltpu.sync_copy(acc.at[pl.ds(row0, npt)],
                    aggp_hbm.at[c, pl.ds(row0, npt)])

  return agg_kernel


def _tc_layer1(degp0_ref, degp1_ref, x_ref, w1_ref, dinv_ref, y1_ref):
  deg = degp0_ref[...] + degp1_ref[...] + 1.0
  dinv = lax.rsqrt(deg)
  dinv_ref[...] = dinv
  xw = jnp.dot(x_ref[...], w1_ref[...], preferred_element_type=jnp.float32)
  y1_ref[...] = xw * dinv[:, 0:1]


def _tc_layer2(aggp0_ref, aggp1_ref, y1_ref, dinv_ref, w2_ref, y2_ref):
  dv = dinv_ref[...][:, 0:1]
  h = jnp.tanh((aggp0_ref[...] + aggp1_ref[...] + y1_ref[...]) * dv)
  y2_ref[...] = jnp.dot(h, w2_ref[...],
                        preferred_element_type=jnp.float32) * dv


def _tc_final(aggp0_ref, aggp1_ref, y2_ref, dinv_ref, out_ref):
  dv = dinv_ref[...][:, 0:1]
  out_ref[...] = (aggp0_ref[...] + aggp1_ref[...] + y2_ref[...]) * dv


def kernel(x, edge_index, W1, W2):
  n, f_in = x.shape
  e = edge_index.shape[1]
  h = W1.shape[1]
  cdim = W2.shape[1]
  cpad = 128
  assert e % CH == 0
  src2 = edge_index[0].reshape(e // CH, CH)
  dst2 = edge_index[1].reshape(e // CH, CH)
  W2p = jnp.zeros((h, cpad), jnp.float32).at[:, :cdim].set(W2)
  np_pad = -(-n // (8 * NS)) * (8 * NS)

  blk = 2000
  assert n % blk == 0
  grid = (n // blk,)
  row_spec = lambda width: pl.BlockSpec((blk, width), lambda i: (i, 0))
  full_spec = lambda r, ccol: pl.BlockSpec((r, ccol), lambda i: (0, 0))

  degp = _make_deg_kernel(np_pad, e)(dst2)

  dinv, y1 = pl.pallas_call(
      _tc_layer1,
      grid=grid,
      in_specs=[row_spec(L), row_spec(L), row_spec(f_in), full_spec(f_in, h)],
      out_specs=[row_spec(L), row_spec(h)],
      out_shape=[
          jax.ShapeDtypeStruct((n, L), jnp.float32),
          jax.ShapeDtypeStruct((n, h), jnp.float32),
      ],
  )(degp[0], degp[1], x, W1)

  aggp1 = _make_agg_kernel(np_pad, e, h)(y1, src2, dst2)

  y2 = pl.pallas_call(
      _tc_layer2,
      grid=grid,
      in_specs=[row_spec(h), row_spec(h), row_spec(h), row_spec(L),
                full_spec(h, cpad)],
      out_specs=row_spec(cpad),
      out_shape=jax.ShapeDtypeStruct((n, cpad), jnp.float32),
  )(aggp1[0], aggp1[1], y1, dinv, W2p)

  aggp2 = _make_agg_kernel(np_pad, e, cpad)(y2, src2, dst2)

  out = pl.pallas_call(
      _tc_final,
      grid=grid,
      in_specs=[row_spec(cpad), row_spec(cpad), row_spec(cpad), row_spec(L)],
      out_specs=row_spec(cpad),
      out_shape=jax.ShapeDtypeStruct((n, cpad), jnp.float32),
  )(aggp2[0], aggp2[1], y2, dinv)

  return out[:, :cdim]
